# Initial kernel scaffold; baseline (speedup 1.0000x reference)
import jax
import jax.numpy as jnp
from jax.experimental import pallas as pl


def kernel(node_features, edge_index, edge_features, nW1, nb1, nW2, nb2, eW1, eb1, eW2, eb2, mW1, mb1, mW2, mb2, uW1, ub1, uW2, ub2, g, bta, rW1, rb1, rW2, rb2):
    raise NotImplementedError("write your pallas kernel here")


# speedup vs baseline: 4.7861x; 4.7861x over previous
import functools

import jax
import jax.numpy as jnp
from jax import lax
from jax.experimental import pallas as pl
from jax.experimental.pallas import tpu as pltpu
from jax.experimental.pallas import tpu_sc as plsc

_N = 10000
_E = 320000
_H = 128
_HH = _H // 2
_L = 3

_NC = 2
_NS = 16
_NW = _NC * _NS
_C = 80

_EPT = _E // _NS
_NCH = _EPT // _C
_EPW = _E // _NW
_NCHD = _EPW // _C

_NP = 10112
_RPS = _NP // _NS
_ZR = 316

_BN = 1264


def _sc_mesh():
    return plsc.VectorSubcoreMesh(
        core_axis_name="c", subcore_axis_name="s",
        num_cores=_NC, num_subcores=_NS)


def _edge_pass(xa2, xb2, srcg, dstg):
    @functools.partial(
        pl.kernel,
        out_type=jax.ShapeDtypeStruct((_NC, _NP, _HH), jnp.float32),
        mesh=_sc_mesh(),
        scratch_types=[
            pltpu.VMEM((_NCH, _C), jnp.int32),
            pltpu.VMEM((_NCH, _C), jnp.int32),
            pltpu.VMEM((_NCH, _C), jnp.int32),
            pltpu.VMEM((_C, _HH), jnp.float32),
            pltpu.VMEM((_C, _HH), jnp.float32),
            pltpu.VMEM((_ZR, _HH), jnp.float32),
            pltpu.VMEM_SHARED((_NP, _HH), jnp.float32),
            pltpu.SemaphoreType.DMA,
            pltpu.SemaphoreType.DMA,
        ],
        compiler_params=pltpu.CompilerParams(use_tc_tiling_on_sc=False),
    )
    def k(xa_hbm, xb_hbm, src_hbm, dst_hbm, out_hbm,
          src_v, dstg_v, dsts_v, arow, brow, zbuf, r_sh, sem_a, sem_b):
        cid = lax.axis_index("c")
        sid = lax.axis_index("s")

        pltpu.sync_copy(src_hbm.at[sid], src_v)
        pltpu.sync_copy(dst_hbm.at[sid], dsts_v)

        def xform(j, carry):
            for cc in range(_C // 16):
                sl = pl.ds(cc * 16, 16)
                s_v = src_v[j, sl]
                src_v[j, sl] = (s_v << 1) | cid
                d_v = dsts_v[j, sl]
                dstg_v[j, sl] = (d_v << 1) | cid
            return carry
        lax.fori_loop(0, _NCH, xform, 0)

        def zstore(t, carry):
            r = t // (_HH // 16)
            cc = (t % (_HH // 16)) * 16
            zbuf[r, pl.ds(cc, 16)] = jnp.zeros((16,), jnp.float32)
            return carry
        lax.fori_loop(0, _ZR * (_HH // 16), zstore, 0)
        for t in range(_RPS // _ZR):
            pltpu.sync_copy(zbuf, r_sh.at[pl.ds(sid * _RPS + t * _ZR, _ZR)])
        plsc.subcore_barrier()

        def chunk(j, carry):
            cpa = pltpu.async_copy(xa_hbm.at[src_v.at[j]], arow, sem_a)
            cpb = pltpu.async_copy(xb_hbm.at[dstg_v.at[j]], brow, sem_b)
            cpa.wait()
            cpb.wait()

            def rows(r, rc):
                for cc in range(_HH // 16):
                    sl = pl.ds(cc * 16, 16)
                    arow[r, sl] = jnp.maximum(arow[r, sl] + brow[r, sl], 0.0)
                return rc
            lax.fori_loop(0, _C, rows, 0)
            pltpu.sync_copy(arow, r_sh.at[dsts_v.at[j]], add=True)
            return carry
        lax.fori_loop(0, _NCH, chunk, 0)

        plsc.subcore_barrier()
        pltpu.sync_copy(r_sh.at[pl.ds(sid * _RPS, _RPS)],
                        out_hbm.at[cid, pl.ds(sid * _RPS, _RPS)])

    return k(xa2, xb2, srcg, dstg)


def _degree(dstg):
    @functools.partial(
        pl.kernel,
        out_type=jax.ShapeDtypeStruct((_NC, _NP, 16), jnp.float32),
        mesh=_sc_mesh(),
        scratch_types=[
            pltpu.VMEM((_NCHD, _C), jnp.int32),
            pltpu.VMEM((_C, 16), jnp.float32),
            pltpu.VMEM((_RPS, 16), jnp.float32),
            pltpu.VMEM_SHARED((_NP, 16), jnp.float32),
        ],
        compiler_params=pltpu.CompilerParams(use_tc_tiling_on_sc=False),
    )
    def k(dst_hbm, out_hbm, dst_v, ones, zbuf, d_sh):
        cid = lax.axis_index("c")
        sid = lax.axis_index("s")
        wid = sid * _NC + cid

        pltpu.sync_copy(dst_hbm.at[wid], dst_v)

        def fill(r, carry):
            ones[r, pl.ds(0, 16)] = jnp.full((16,), 1.0, jnp.float32)
            return carry
        lax.fori_loop(0, _C, fill, 0)

        def zstore(r, carry):
            zbuf[r, pl.ds(0, 16)] = jnp.zeros((16,), jnp.float32)
            return carry
        lax.fori_loop(0, _RPS, zstore, 0)
        pltpu.sync_copy(zbuf, d_sh.at[pl.ds(sid * _RPS, _RPS)])
        plsc.subcore_barrier()

        def chunk(j, carry):
            pltpu.sync_copy(ones, d_sh.at[dst_v.at[j]], add=True)
            return carry
        lax.fori_loop(0, _NCHD, chunk, 0)

        plsc.subcore_barrier()
        pltpu.sync_copy(d_sh.at[pl.ds(sid * _RPS, _RPS)],
                        out_hbm.at[cid, pl.ds(sid * _RPS, _RPS)])

    return k(dstg)


def _full(shape):
    return pl.BlockSpec(shape, lambda i: (0,) * len(shape))


def _blk(w=_H):
    return pl.BlockSpec((_BN, w), lambda i: (i, 0))


def _encode(nf, nW1, nb1, nW2, nb2):
    nd = nf.shape[1]

    def body(nf_r, w1, b1, w2, b2, x_o):
        x_o[...] = (jnp.maximum(nf_r[...] @ w1[...] + b1[...], 0.0)
                    @ w2[...] + b2[...])

    return pl.pallas_call(
        body,
        grid=(_NP // _BN,),
        in_specs=[_blk(nd)] + [_full(w.shape) for w in (nW1, nb1, nW2, nb2)],
        out_specs=_blk(),
        out_shape=jax.ShapeDtypeStruct((_NP, _H), jnp.float32),
    )(nf, nW1, nb1, nW2, nb2)


def _precompute(x, a_i, b_i, mb_i):
    def body(x_r, a_r, b_r, mb, xa_o, xb_o):
        x_v = x_r[...]
        xa_o[...] = x_v @ a_r[...]
        xb_o[...] = x_v @ b_r[...] + mb[...]

    o = jax.ShapeDtypeStruct((_NP, _H), jnp.float32)
    return pl.pallas_call(
        body,
        grid=(_NP // _BN,),
        in_specs=[_blk()] + [_full(w.shape) for w in (a_i, b_i, mb_i)],
        out_specs=[_blk(), _blk()],
        out_shape=[o, o],
    )(x, a_i, b_i, mb_i)


def _update(x, rl, rr, deg16, mw2l, mw2r, mb2i, uw1a, uw1b, ub1i, uW2i, ub2i,
            gi, bi):
    def body(x_r, rl_r, rr_r, dg_r, w2l, w2r, mb2, w1a, w1b, b1, w2, b2,
             gg, bb, x_o):
        x_v = x_r[...]
        cnt = dg_r[...][:, 0:1]
        agg = rl_r[...] @ w2l[...] + rr_r[...] @ w2r[...] + cnt * mb2[...]
        h = jnp.maximum(x_v @ w1a[...] + agg @ w1b[...] + b1[...], 0.0)
        u = h @ w2[...] + b2[...]
        y = x_v + u
        mu = jnp.mean(y, axis=-1, keepdims=True)
        var = jnp.mean((y - mu) ** 2, axis=-1, keepdims=True)
        x_o[...] = (y - mu) / jnp.sqrt(var + 1e-5) * gg[...] + bb[...]

    ws = (mw2l, mw2r, mb2i, uw1a, uw1b, ub1i, uW2i, ub2i, gi, bi)
    return pl.pallas_call(
        body,
        grid=(_NP // _BN,),
        in_specs=([_blk(), _blk(_HH), _blk(_HH), _blk(16)]
                  + [_full(w.shape) for w in ws]),
        out_specs=_blk(),
        out_shape=jax.ShapeDtypeStruct((_NP, _H), jnp.float32),
    )(x, rl, rr, deg16, *ws)


def _readout(x, rW1, rb1, rW2, rb2):
    def body(x_r, w1, b1, w2, b2, o_r):
        ge = jnp.mean(x_r[...][:_N], axis=0, keepdims=True)
        ge8 = jnp.broadcast_to(ge, (8, _H))
        o_r[...] = jnp.maximum(ge8 @ w1[...] + b1[...], 0.0) @ w2[...] + b2[...]

    return pl.pallas_call(
        body,
        out_shape=jax.ShapeDtypeStruct((8, _H), jnp.float32),
    )(x, rW1, rb1, rW2, rb2)


def kernel(node_features, edge_index, edge_features, nW1, nb1, nW2, nb2,
           eW1, eb1, eW2, eb2, mW1, mb1, mW2, mb2, uW1, ub1, uW2, ub2,
           g, bta, rW1, rb1, rW2, rb2):
    del edge_features, eW1, eb1, eW2, eb2

    nf = jnp.pad(node_features, ((0, _NP - _N), (0, 0)))
    srcg = edge_index[0].reshape(_NS, _NCH, _C)
    dstg = edge_index[1].reshape(_NS, _NCH, _C)
    dstw = edge_index[1].reshape(_NW, _NCHD, _C)

    def row(v):
        return v.reshape(1, -1)

    deg2 = _degree(dstw)
    deg16 = deg2[0] + deg2[1]

    x = _encode(nf, nW1, row(nb1), nW2, row(nb2))

    xs = (mW1[:, :_H, :], mW1[:, _H:, :], mb1[:, None, :],
          mW2[:, :_HH, :], mW2[:, _HH:, :], mb2[:, None, :],
          uW1[:, :_H, :], uW1[:, _H:, :], ub1[:, None, :],
          uW2, ub2[:, None, :], g[:, None, :], bta[:, None, :])

    def step(x_c, ws):
        (a_i, b_i, mb_i, w2l, w2r, mb2_i, u1a, u1b, ub1_i, uw2_i, ub2_i,
         g_i, bta_i) = ws
        xa, xb = _precompute(x_c, a_i, b_i, mb_i)
        r2 = _edge_pass(xa.reshape(2 * _NP, _HH), xb.reshape(2 * _NP, _HH),
                        srcg, dstg)
        x_n = _update(x_c, r2[0], r2[1], deg16, w2l, w2r, mb2_i,
                      u1a, u1b, ub1_i, uw2_i, ub2_i, g_i, bta_i)
        return x_n, None

    x, _ = lax.scan(step, x, xs)

    out8 = _readout(x, rW1, row(rb1), rW2, row(rb2))
    return out8[0]


# speedup vs baseline: 8.5234x; 1.7808x over previous
import functools

import jax
import jax.numpy as jnp
from jax import lax
from jax.experimental import pallas as pl
from jax.experimental.pallas import tpu as pltpu
from jax.experimental.pallas import tpu_sc as plsc

_N = 10000
_E = 320000
_H = 128
_HH = _H // 2
_L = 3

_NC = 2
_NS = 16
_NW = _NC * _NS
_C = 80

_EPT = _E // _NS
_NCH = _EPT // _C
_EPW = _E // _NW
_NCHD = _EPW // _C

_NB = 5
_NP = 10112
_RPS = _NP // _NS
_ZR = 158

_BN = 1264


def _sc_mesh():
    return plsc.VectorSubcoreMesh(
        core_axis_name="c", subcore_axis_name="s",
        num_cores=_NC, num_subcores=_NS)


def _edge_pass(xa3, xb3, srcg, dstg):
    @functools.partial(
        pl.kernel,
        out_type=jax.ShapeDtypeStruct((_NC, _NP, _HH), jnp.float32),
        mesh=_sc_mesh(),
        scratch_types=[
            pltpu.VMEM((_NCH, _C), jnp.int32),
            pltpu.VMEM((_NCH, _C), jnp.int32),
            pltpu.VMEM((_C, _HH), jnp.float32),
            pltpu.VMEM((_C, _HH), jnp.float32),
            pltpu.VMEM((_C, _HH), jnp.float32),
            pltpu.VMEM((_C, _HH), jnp.float32),
            pltpu.VMEM((_C, _HH), jnp.float32),
            pltpu.VMEM((_C, _HH), jnp.float32),
            pltpu.VMEM((_ZR, _HH), jnp.float32),
            pltpu.VMEM_SHARED((_NP, _HH), jnp.float32),
            pltpu.SemaphoreType.DMA,
            pltpu.SemaphoreType.DMA,
            pltpu.SemaphoreType.DMA,
            pltpu.SemaphoreType.DMA,
            pltpu.SemaphoreType.DMA,
            pltpu.SemaphoreType.DMA,
        ],
        compiler_params=pltpu.CompilerParams(use_tc_tiling_on_sc=False),
    )
    def k(xa_hbm, xb_hbm, src_hbm, dst_hbm, out_hbm,
          src_v, dst_v, ar0, ar1, ar2, br0, br1, br2, zbuf, r_sh,
          sg0, sg1, sg2, ss0, ss1, ss2):
        cid = lax.axis_index("c")
        sid = lax.axis_index("s")
        ar = [ar0, ar1, ar2]
        br = [br0, br1, br2]
        sg = [sg0, sg1, sg2]
        ss = [ss0, ss1, ss2]

        pltpu.sync_copy(src_hbm.at[sid], src_v)
        pltpu.sync_copy(dst_hbm.at[sid], dst_v)

        def zstore(t, carry):
            r = t // (_HH // 16)
            cc = (t % (_HH // 16)) * 16
            zbuf[r, pl.ds(cc, 16)] = jnp.zeros((16,), jnp.float32)
            return carry
        lax.fori_loop(0, _ZR * (_HH // 16), zstore, 0)
        for t in range(_RPS // _ZR):
            pltpu.sync_copy(zbuf, r_sh.at[pl.ds(sid * _RPS + t * _ZR, _ZR)])
        plsc.subcore_barrier()

        def gissue(j, b):
            pltpu.async_copy(xa_hbm.at[cid].at[src_v.at[j]], ar[b], sg[b])
            pltpu.async_copy(xb_hbm.at[cid].at[dst_v.at[j]], br[b], sg[b])

        def gwait(j, b):
            pltpu.make_async_copy(xa_hbm.at[cid].at[src_v.at[j]],
                                  ar[b], sg[b]).wait()
            pltpu.make_async_copy(xb_hbm.at[cid].at[dst_v.at[j]],
                                  br[b], sg[b]).wait()

        def swait(j, b):
            pltpu.make_async_copy(ar[b], r_sh.at[dst_v.at[j]], ss[b]).wait()

        gissue(0, 0)
        gissue(1, 1)

        def body(j, b):
            bn = (b + 2) % 3

            @pl.when(jnp.logical_and(j >= 1, j + 2 < _NCH))
            def _():
                swait(j - 1, bn)

            @pl.when(j + 2 < _NCH)
            def _():
                gissue(j + 2, bn)

            gwait(j, b)

            def rows(r8, rc):
                for rr in range(8):
                    for cc in range(_HH // 16):
                        sl = pl.ds(cc * 16, 16)
                        r = r8 * 8 + rr
                        ar[b][r, sl] = jnp.maximum(
                            ar[b][r, sl] + br[b][r, sl], 0.0)
                return rc
            lax.fori_loop(0, _C // 8, rows, 0)

            pltpu.async_copy(ar[b], r_sh.at[dst_v.at[j]], ss[b],
                             add=True)

        def round_(j0, carry):
            j3 = j0 * 3
            for b in range(3):
                body(j3 + b, b)
            return carry
        lax.fori_loop(0, (_NCH - 1) // 3, round_, 0)
        body(_NCH - 1, (_NCH - 1) % 3)

        for b in range(3):
            swait(_NCH - 3 + b, (_NCH - 3 + b) % 3)

        plsc.subcore_barrier()
        pltpu.sync_copy(r_sh.at[pl.ds(sid * _RPS, _RPS)],
                        out_hbm.at[cid, pl.ds(sid * _RPS, _RPS)])

    return k(xa3, xb3, srcg, dstg)


def _degree(dstg):
    @functools.partial(
        pl.kernel,
        out_type=jax.ShapeDtypeStruct((_NC, _NP, 16), jnp.float32),
        mesh=_sc_mesh(),
        scratch_types=[
            pltpu.VMEM((_NCHD, _C), jnp.int32),
            pltpu.VMEM((_C, 16), jnp.float32),
            pltpu.VMEM((_RPS, 16), jnp.float32),
            pltpu.VMEM_SHARED((_NP, 16), jnp.float32),
        ],
        compiler_params=pltpu.CompilerParams(use_tc_tiling_on_sc=False),
    )
    def k(dst_hbm, out_hbm, dst_v, ones, zbuf, d_sh):
        cid = lax.axis_index("c")
        sid = lax.axis_index("s")
        wid = sid * _NC + cid

        pltpu.sync_copy(dst_hbm.at[wid], dst_v)

        def fill(r, carry):
            ones[r, pl.ds(0, 16)] = jnp.full((16,), 1.0, jnp.float32)
            return carry
        lax.fori_loop(0, _C, fill, 0)

        def zstore(r, carry):
            zbuf[r, pl.ds(0, 16)] = jnp.zeros((16,), jnp.float32)
            return carry
        lax.fori_loop(0, _RPS, zstore, 0)
        pltpu.sync_copy(zbuf, d_sh.at[pl.ds(sid * _RPS, _RPS)])
        plsc.subcore_barrier()

        def chunk(j, carry):
            pltpu.sync_copy(ones, d_sh.at[dst_v.at[j]], add=True)
            return carry
        lax.fori_loop(0, _NCHD, chunk, 0)

        plsc.subcore_barrier()
        pltpu.sync_copy(d_sh.at[pl.ds(sid * _RPS, _RPS)],
                        out_hbm.at[cid, pl.ds(sid * _RPS, _RPS)])

    return k(dstg)


def _full(shape):
    return pl.BlockSpec(shape, lambda i: (0,) * len(shape))


def _blk(w=_H):
    return pl.BlockSpec((_BN, w), lambda i: (i, 0))


def _encode(nf, nW1, nb1, nW2, nb2):
    nd = nf.shape[1]

    def body(nf_r, w1, b1, w2, b2, x_o):
        x_o[...] = (jnp.maximum(nf_r[...] @ w1[...] + b1[...], 0.0)
                    @ w2[...] + b2[...])

    return pl.pallas_call(
        body,
        grid=(_NP // _BN,),
        in_specs=[_blk(nd)] + [_full(w.shape) for w in (nW1, nb1, nW2, nb2)],
        out_specs=_blk(),
        out_shape=jax.ShapeDtypeStruct((_NP, _H), jnp.float32),
    )(nf, nW1, nb1, nW2, nb2)


def _precompute(x, a_i, b_i, mb_i):
    def body(x_r, a_r, b_r, mb, xa_o, xb_o):
        x_v = x_r[...]
        xa_o[0] = x_v @ a_r[0]
        xb_o[0] = x_v @ b_r[0] + mb[0]

    o = jax.ShapeDtypeStruct((_NC, _NP, _HH), jnp.float32)
    hblk = pl.BlockSpec((1, _BN, _HH), lambda i, c: (c, i, 0))
    return pl.pallas_call(
        body,
        grid=(_NP // _BN, _NC),
        in_specs=[pl.BlockSpec((_BN, _H), lambda i, c: (i, 0)),
                  pl.BlockSpec((1, _H, _HH), lambda i, c: (c, 0, 0)),
                  pl.BlockSpec((1, _H, _HH), lambda i, c: (c, 0, 0)),
                  pl.BlockSpec((1, 1, _HH), lambda i, c: (c, 0, 0))],
        out_specs=[hblk, hblk],
        out_shape=[o, o],
    )(x, a_i, b_i, mb_i)


def _update(x, rl, rr, deg16, mw2l, mw2r, mb2i, uw1a, uw1b, ub1i, uW2i, ub2i,
            gi, bi):
    def body(x_r, rl_r, rr_r, dg_r, w2l, w2r, mb2, w1a, w1b, b1, w2, b2,
             gg, bb, x_o):
        x_v = x_r[...]
        cnt = dg_r[...][:, 0:1]
        agg = rl_r[...] @ w2l[...] + rr_r[...] @ w2r[...] + cnt * mb2[...]
        h = jnp.maximum(x_v @ w1a[...] + agg @ w1b[...] + b1[...], 0.0)
        u = h @ w2[...] + b2[...]
        y = x_v + u
        mu = jnp.mean(y, axis=-1, keepdims=True)
        var = jnp.mean((y - mu) ** 2, axis=-1, keepdims=True)
        x_o[...] = (y - mu) / jnp.sqrt(var + 1e-5) * gg[...] + bb[...]

    ws = (mw2l, mw2r, mb2i, uw1a, uw1b, ub1i, uW2i, ub2i, gi, bi)
    return pl.pallas_call(
        body,
        grid=(_NP // _BN,),
        in_specs=([_blk(), _blk(_HH), _blk(_HH), _blk(16)]
                  + [_full(w.shape) for w in ws]),
        out_specs=_blk(),
        out_shape=jax.ShapeDtypeStruct((_NP, _H), jnp.float32),
    )(x, rl, rr, deg16, *ws)


def _readout(x, rW1, rb1, rW2, rb2):
    def body(x_r, w1, b1, w2, b2, o_r):
        ge = jnp.mean(x_r[...][:_N], axis=0, keepdims=True)
        ge8 = jnp.broadcast_to(ge, (8, _H))
        o_r[...] = jnp.maximum(ge8 @ w1[...] + b1[...], 0.0) @ w2[...] + b2[...]

    return pl.pallas_call(
        body,
        out_shape=jax.ShapeDtypeStruct((8, _H), jnp.float32),
    )(x, rW1, rb1, rW2, rb2)


def kernel(node_features, edge_index, edge_features, nW1, nb1, nW2, nb2,
           eW1, eb1, eW2, eb2, mW1, mb1, mW2, mb2, uW1, ub1, uW2, ub2,
           g, bta, rW1, rb1, rW2, rb2):
    del edge_features, eW1, eb1, eW2, eb2

    nf = jnp.pad(node_features, ((0, _NP - _N), (0, 0)))
    srcg = edge_index[0].reshape(_NS, _NCH, _C)
    dstg = edge_index[1].reshape(_NS, _NCH, _C)
    dstw = edge_index[1].reshape(_NW, _NCHD, _C)

    def row(v):
        return v.reshape(1, -1)

    deg2 = _degree(dstw)
    deg16 = deg2[0] + deg2[1]

    x = _encode(nf, nW1, row(nb1), nW2, row(nb2))

    def csplit(w):
        return w.reshape(w.shape[0], w.shape[1], _NC, _HH).transpose(0, 2, 1, 3)

    xs = (csplit(mW1[:, :_H, :]), csplit(mW1[:, _H:, :]),
          csplit(mb1[:, None, :]),
          mW2[:, :_HH, :], mW2[:, _HH:, :], mb2[:, None, :],
          uW1[:, :_H, :], uW1[:, _H:, :], ub1[:, None, :],
          uW2, ub2[:, None, :], g[:, None, :], bta[:, None, :])

    def step(x_c, ws):
        (a_i, b_i, mb_i, w2l, w2r, mb2_i, u1a, u1b, ub1_i, uw2_i, ub2_i,
         g_i, bta_i) = ws
        xa3, xb3 = _precompute(x_c, a_i, b_i, mb_i)
        r2 = _edge_pass(xa3, xb3, srcg, dstg)
        x_n = _update(x_c, r2[0], r2[1], deg16, w2l, w2r, mb2_i,
                      u1a, u1b, ub1_i, uw2_i, ub2_i, g_i, bta_i)
        return x_n, None

    x, _ = lax.scan(step, x, xs)

    out8 = _readout(x, rW1, row(rb1), rW2, row(rb2))
    return out8[0]


# speedup vs baseline: 8.9954x; 1.0554x over previous
import functools

import jax
import jax.numpy as jnp
from jax import lax
from jax.experimental import pallas as pl
from jax.experimental.pallas import tpu as pltpu
from jax.experimental.pallas import tpu_sc as plsc

_N = 10000
_E = 320000
_H = 128
_HH = _H // 2
_L = 3

_NC = 2
_NS = 16
_NW = _NC * _NS
_C = 80

_EPT = _E // _NS
_NCH = _EPT // _C
_EPW = _E // _NW
_NCHD = _EPW // _C

_NB = 5
_NP = 10112
_RPS = _NP // _NS
_ZR = 158

_BN = 1264


def _sc_mesh():
    return plsc.VectorSubcoreMesh(
        core_axis_name="c", subcore_axis_name="s",
        num_cores=_NC, num_subcores=_NS)


def _edge_pass(xa3, xb3, srcg, dstg):
    @functools.partial(
        pl.kernel,
        out_type=jax.ShapeDtypeStruct((_NC, _NP, _HH), jnp.float32),
        mesh=_sc_mesh(),
        scratch_types=[
            pltpu.VMEM((_NCH, _C), jnp.int32),
            pltpu.VMEM((_NCH, _C), jnp.int32),
            pltpu.VMEM((_C, _HH), jnp.float32),
            pltpu.VMEM((_C, _HH), jnp.float32),
            pltpu.VMEM((_C, _HH), jnp.float32),
            pltpu.VMEM((_C, _HH), jnp.float32),
            pltpu.VMEM((_C, _HH), jnp.float32),
            pltpu.VMEM((_C, _HH), jnp.float32),
            pltpu.VMEM((_ZR, _HH), jnp.float32),
            pltpu.VMEM_SHARED((_NP, _HH), jnp.float32),
            pltpu.SemaphoreType.DMA,
            pltpu.SemaphoreType.DMA,
            pltpu.SemaphoreType.DMA,
            pltpu.SemaphoreType.DMA,
            pltpu.SemaphoreType.DMA,
            pltpu.SemaphoreType.DMA,
        ],
        compiler_params=pltpu.CompilerParams(use_tc_tiling_on_sc=False),
    )
    def k(xa_hbm, xb_hbm, src_hbm, dst_hbm, out_hbm,
          src_v, dst_v, ar0, ar1, ar2, br0, br1, br2, zbuf, r_sh,
          sg0, sg1, sg2, ss0, ss1, ss2):
        cid = lax.axis_index("c")
        sid = lax.axis_index("s")
        ar = [ar0, ar1, ar2]
        br = [br0, br1, br2]
        sg = [sg0, sg1, sg2]
        ss = [ss0, ss1, ss2]

        pltpu.sync_copy(src_hbm.at[sid], src_v)
        pltpu.sync_copy(dst_hbm.at[sid], dst_v)

        def zstore(t, carry):
            r = t // (_HH // 16)
            cc = (t % (_HH // 16)) * 16
            zbuf[r, pl.ds(cc, 16)] = jnp.zeros((16,), jnp.float32)
            return carry
        lax.fori_loop(0, _ZR * (_HH // 16), zstore, 0)
        for t in range(_RPS // _ZR):
            pltpu.sync_copy(zbuf, r_sh.at[pl.ds(sid * _RPS + t * _ZR, _ZR)])
        plsc.subcore_barrier()

        def gissue(j, b):
            pltpu.async_copy(xa_hbm.at[cid].at[src_v.at[j]], ar[b], sg[b])
            pltpu.async_copy(xb_hbm.at[cid].at[dst_v.at[j]], br[b], sg[b])

        def gwait(j, b):
            pltpu.make_async_copy(xa_hbm.at[cid].at[src_v.at[j]],
                                  ar[b], sg[b]).wait()
            pltpu.make_async_copy(xb_hbm.at[cid].at[dst_v.at[j]],
                                  br[b], sg[b]).wait()

        def swait(j, b):
            pltpu.make_async_copy(ar[b], r_sh.at[dst_v.at[j]], ss[b]).wait()

        gissue(0, 0)
        gissue(1, 1)

        def body(j, b):
            bn = (b + 2) % 3

            @pl.when(jnp.logical_and(j >= 1, j + 2 < _NCH))
            def _():
                swait(j - 1, bn)

            @pl.when(j + 2 < _NCH)
            def _():
                gissue(j + 2, bn)

            gwait(j, b)

            def rows(r8, rc):
                for rr in range(8):
                    for cc in range(_HH // 16):
                        sl = pl.ds(cc * 16, 16)
                        r = r8 * 8 + rr
                        ar[b][r, sl] = jnp.maximum(
                            ar[b][r, sl] + br[b][r, sl], 0.0)
                return rc
            lax.fori_loop(0, _C // 8, rows, 0)

            pltpu.async_copy(ar[b], r_sh.at[dst_v.at[j]], ss[b],
                             add=True)

        def round_(j0, carry):
            j3 = j0 * 3
            for b in range(3):
                body(j3 + b, b)
            return carry
        lax.fori_loop(0, (_NCH - 1) // 3, round_, 0)
        body(_NCH - 1, (_NCH - 1) % 3)

        for b in range(3):
            swait(_NCH - 3 + b, (_NCH - 3 + b) % 3)

        plsc.subcore_barrier()
        pltpu.sync_copy(r_sh.at[pl.ds(sid * _RPS, _RPS)],
                        out_hbm.at[cid, pl.ds(sid * _RPS, _RPS)])

    return k(xa3, xb3, srcg, dstg)


def _degree(dstg):
    @functools.partial(
        pl.kernel,
        out_type=jax.ShapeDtypeStruct((_NC, _NP, 16), jnp.float32),
        mesh=_sc_mesh(),
        scratch_types=[
            pltpu.VMEM((_NCHD, _C), jnp.int32),
            pltpu.VMEM((_C, 16), jnp.float32),
            pltpu.VMEM((_RPS, 16), jnp.float32),
            pltpu.VMEM_SHARED((_NP, 16), jnp.float32),
        ],
        compiler_params=pltpu.CompilerParams(use_tc_tiling_on_sc=False),
    )
    def k(dst_hbm, out_hbm, dst_v, ones, zbuf, d_sh):
        cid = lax.axis_index("c")
        sid = lax.axis_index("s")
        wid = sid * _NC + cid

        pltpu.sync_copy(dst_hbm.at[wid], dst_v)

        def fill(r, carry):
            ones[r, pl.ds(0, 16)] = jnp.full((16,), 1.0, jnp.float32)
            return carry
        lax.fori_loop(0, _C, fill, 0)

        def zstore(r, carry):
            zbuf[r, pl.ds(0, 16)] = jnp.zeros((16,), jnp.float32)
            return carry
        lax.fori_loop(0, _RPS, zstore, 0)
        pltpu.sync_copy(zbuf, d_sh.at[pl.ds(sid * _RPS, _RPS)])
        plsc.subcore_barrier()

        def chunk(j, carry):
            pltpu.sync_copy(ones, d_sh.at[dst_v.at[j]], add=True)
            return carry
        lax.fori_loop(0, _NCHD, chunk, 0)

        plsc.subcore_barrier()
        pltpu.sync_copy(d_sh.at[pl.ds(sid * _RPS, _RPS)],
                        out_hbm.at[cid, pl.ds(sid * _RPS, _RPS)])

    return k(dstg)


def _full(shape):
    return pl.BlockSpec(shape, lambda i: (0,) * len(shape))


def _blk(w=_H):
    return pl.BlockSpec((_BN, w), lambda i: (i, 0))


def _encode(nf, nW1, nb1, nW2, nb2, a0, b0, mb0):
    nd = nf.shape[1]

    def body(nf_r, w1, b1, w2, b2, a_r, b_r, mb, x_o, xa_o, xb_o):
        x = (jnp.maximum(nf_r[...] @ w1[...] + b1[...], 0.0)
             @ w2[...] + b2[...])
        x_o[...] = x
        for c in range(_NC):
            xa_o[c] = x @ a_r[c]
            xb_o[c] = x @ b_r[c] + mb[c]

    o3 = jax.ShapeDtypeStruct((_NC, _NP, _HH), jnp.float32)
    hblk = pl.BlockSpec((_NC, _BN, _HH), lambda i: (0, i, 0))
    return pl.pallas_call(
        body,
        grid=(_NP // _BN,),
        in_specs=[_blk(nd)] + [_full(w.shape) for w in
                               (nW1, nb1, nW2, nb2, a0, b0, mb0)],
        out_specs=[_blk(), hblk, hblk],
        out_shape=[jax.ShapeDtypeStruct((_NP, _H), jnp.float32), o3, o3],
    )(nf, nW1, nb1, nW2, nb2, a0, b0, mb0)


def _update(x, rl, rr, deg16, mw2l, mw2r, mb2i, uw1a, uw1b, ub1i, uW2i, ub2i,
            gi, bi, a_n, b_n, mb_n):
    def body(x_r, rl_r, rr_r, dg_r, w2l, w2r, mb2, w1a, w1b, b1, w2, b2,
             gg, bb, a_r, b_r, mbn, x_o, xa_o, xb_o):
        x_v = x_r[...]
        cnt = dg_r[...][:, 0:1]
        agg = rl_r[...] @ w2l[...] + rr_r[...] @ w2r[...] + cnt * mb2[...]
        h = jnp.maximum(x_v @ w1a[...] + agg @ w1b[...] + b1[...], 0.0)
        u = h @ w2[...] + b2[...]
        y = x_v + u
        mu = jnp.mean(y, axis=-1, keepdims=True)
        var = jnp.mean((y - mu) ** 2, axis=-1, keepdims=True)
        xn = (y - mu) / jnp.sqrt(var + 1e-5) * gg[...] + bb[...]
        x_o[...] = xn
        for c in range(_NC):
            xa_o[c] = xn @ a_r[c]
            xb_o[c] = xn @ b_r[c] + mbn[c]

    ws = (mw2l, mw2r, mb2i, uw1a, uw1b, ub1i, uW2i, ub2i, gi, bi,
          a_n, b_n, mb_n)
    o3 = jax.ShapeDtypeStruct((_NC, _NP, _HH), jnp.float32)
    hblk = pl.BlockSpec((_NC, _BN, _HH), lambda i: (0, i, 0))
    return pl.pallas_call(
        body,
        grid=(_NP // _BN,),
        in_specs=([_blk(), _blk(_HH), _blk(_HH), _blk(16)]
                  + [_full(w.shape) for w in ws]),
        out_specs=[_blk(), hblk, hblk],
        out_shape=[jax.ShapeDtypeStruct((_NP, _H), jnp.float32), o3, o3],
    )(x, rl, rr, deg16, *ws)


def _readout(x, rW1, rb1, rW2, rb2):
    def body(x_r, w1, b1, w2, b2, o_r):
        ge = jnp.mean(x_r[...][:_N], axis=0, keepdims=True)
        ge8 = jnp.broadcast_to(ge, (8, _H))
        o_r[...] = jnp.maximum(ge8 @ w1[...] + b1[...], 0.0) @ w2[...] + b2[...]

    return pl.pallas_call(
        body,
        out_shape=jax.ShapeDtypeStruct((8, _H), jnp.float32),
    )(x, rW1, rb1, rW2, rb2)


def kernel(node_features, edge_index, edge_features, nW1, nb1, nW2, nb2,
           eW1, eb1, eW2, eb2, mW1, mb1, mW2, mb2, uW1, ub1, uW2, ub2,
           g, bta, rW1, rb1, rW2, rb2):
    del edge_features, eW1, eb1, eW2, eb2

    nf = jnp.pad(node_features, ((0, _NP - _N), (0, 0)))
    srcg = edge_index[0].reshape(_NS, _NCH, _C)
    dstg = edge_index[1].reshape(_NS, _NCH, _C)
    dstw = edge_index[1].reshape(_NW, _NCHD, _C)

    def row(v):
        return v.reshape(1, -1)

    deg2 = _degree(dstw)
    deg16 = deg2[0] + deg2[1]

    def csplit(w):
        return w.reshape(w.shape[0], w.shape[1], _NC, _HH).transpose(0, 2, 1, 3)

    a_c = csplit(mW1[:, :_H, :])
    b_c = csplit(mW1[:, _H:, :])
    mb_c = csplit(mb1[:, None, :])

    x, xa3, xb3 = _encode(nf, nW1, row(nb1), nW2, row(nb2),
                          a_c[0], b_c[0], mb_c[0])

    def shift(w):
        return jnp.concatenate([w[1:], w[-1:]], axis=0)

    xs = (mW2[:, :_HH, :], mW2[:, _HH:, :], mb2[:, None, :],
          uW1[:, :_H, :], uW1[:, _H:, :], ub1[:, None, :],
          uW2, ub2[:, None, :], g[:, None, :], bta[:, None, :],
          shift(a_c), shift(b_c), shift(mb_c))

    def step(carry, ws):
        x_c, xa_c, xb_c = carry
        (w2l, w2r, mb2_i, u1a, u1b, ub1_i, uw2_i, ub2_i, g_i, bta_i,
         a_n, b_n, mbn) = ws
        r2 = _edge_pass(xa_c, xb_c, srcg, dstg)
        x_n, xa_n, xb_n = _update(x_c, r2[0], r2[1], deg16, w2l, w2r, mb2_i,
                                  u1a, u1b, ub1_i, uw2_i, ub2_i, g_i, bta_i,
                                  a_n, b_n, mbn)
        return (x_n, xa_n, xb_n), None

    (x, _, _), _ = lax.scan(step, (x, xa3, xb3), xs)

    out8 = _readout(x, rW1, row(rb1), rW2, row(rb2))
    return out8[0]


# speedup vs baseline: 9.0006x; 1.0006x over previous
import functools

import jax
import jax.numpy as jnp
from jax import lax
from jax.experimental import pallas as pl
from jax.experimental.pallas import tpu as pltpu
from jax.experimental.pallas import tpu_sc as plsc

_N = 10000
_E = 320000
_H = 128
_HH = _H // 2
_L = 3

_NC = 2
_NS = 16
_NW = _NC * _NS
_C = 80

_EPT = _E // _NS
_NCH = _EPT // _C
_EPW = _E // _NW
_NCHD = _EPW // _C

_NB = 5
_NP = 10112
_RPS = _NP // _NS
_ZR = 158

_BN = 1264


def _sc_mesh():
    return plsc.VectorSubcoreMesh(
        core_axis_name="c", subcore_axis_name="s",
        num_cores=_NC, num_subcores=_NS)


def _edge_pass(xa3, xb3, srcg, dstg):
    @functools.partial(
        pl.kernel,
        out_type=jax.ShapeDtypeStruct((_NC, _NP, _HH), jnp.float32),
        mesh=_sc_mesh(),
        scratch_types=[
            pltpu.VMEM((_NCH, _C), jnp.int32),
            pltpu.VMEM((_NCH, _C), jnp.int32),
            pltpu.VMEM((_C, _HH), jnp.float32),
            pltpu.VMEM((_C, _HH), jnp.float32),
            pltpu.VMEM((_C, _HH), jnp.float32),
            pltpu.VMEM((_C, _HH), jnp.float32),
            pltpu.VMEM((_C, _HH), jnp.float32),
            pltpu.VMEM((_C, _HH), jnp.float32),
            pltpu.VMEM((_ZR, _HH), jnp.float32),
            pltpu.VMEM_SHARED((_NP, _HH), jnp.float32),
            pltpu.SemaphoreType.DMA,
            pltpu.SemaphoreType.DMA,
            pltpu.SemaphoreType.DMA,
            pltpu.SemaphoreType.DMA,
            pltpu.SemaphoreType.DMA,
            pltpu.SemaphoreType.DMA,
        ],
        compiler_params=pltpu.CompilerParams(use_tc_tiling_on_sc=False),
    )
    def k(xa_hbm, xb_hbm, src_hbm, dst_hbm, out_hbm,
          src_v, dst_v, ar0, ar1, ar2, br0, br1, br2, zbuf, r_sh,
          sg0, sg1, sg2, ss0, ss1, ss2):
        cid = lax.axis_index("c")
        sid = lax.axis_index("s")
        ar = [ar0, ar1, ar2]
        br = [br0, br1, br2]
        sg = [sg0, sg1, sg2]
        ss = [ss0, ss1, ss2]

        pltpu.sync_copy(src_hbm.at[sid], src_v)
        pltpu.sync_copy(dst_hbm.at[sid], dst_v)

        def zstore(t, carry):
            r = t // (_HH // 16)
            cc = (t % (_HH // 16)) * 16
            zbuf[r, pl.ds(cc, 16)] = jnp.zeros((16,), jnp.float32)
            return carry
        lax.fori_loop(0, _ZR * (_HH // 16), zstore, 0)
        for t in range(_RPS // _ZR):
            pltpu.sync_copy(zbuf, r_sh.at[pl.ds(sid * _RPS + t * _ZR, _ZR)])
        plsc.subcore_barrier()

        def gissue(j, b):
            pltpu.async_copy(xa_hbm.at[cid].at[src_v.at[j]], ar[b], sg[b])
            pltpu.async_copy(xb_hbm.at[cid].at[dst_v.at[j]], br[b], sg[b])

        def gwait(j, b):
            pltpu.make_async_copy(xa_hbm.at[cid].at[src_v.at[j]],
                                  ar[b], sg[b]).wait()
            pltpu.make_async_copy(xb_hbm.at[cid].at[dst_v.at[j]],
                                  br[b], sg[b]).wait()

        def swait(j, b):
            pltpu.make_async_copy(ar[b], r_sh.at[dst_v.at[j]], ss[b]).wait()

        gissue(0, 0)
        gissue(1, 1)

        def body(j, b):
            bn = (b + 2) % 3

            @pl.when(jnp.logical_and(j >= 1, j + 2 < _NCH))
            def _():
                swait(j - 1, bn)

            @pl.when(j + 2 < _NCH)
            def _():
                gissue(j + 2, bn)

            gwait(j, b)

            def rows(r8, rc):
                for rr in range(8):
                    for cc in range(_HH // 16):
                        sl = pl.ds(cc * 16, 16)
                        r = r8 * 8 + rr
                        ar[b][r, sl] = jnp.maximum(
                            ar[b][r, sl] + br[b][r, sl], 0.0)
                return rc
            lax.fori_loop(0, _C // 8, rows, 0)

            pltpu.async_copy(ar[b], r_sh.at[dst_v.at[j]], ss[b],
                             add=True)

        def round_(j0, carry):
            j3 = j0 * 3
            for b in range(3):
                body(j3 + b, b)
            return carry
        lax.fori_loop(0, (_NCH - 1) // 3, round_, 0)
        body(_NCH - 1, (_NCH - 1) % 3)

        for b in range(3):
            swait(_NCH - 3 + b, (_NCH - 3 + b) % 3)

        plsc.subcore_barrier()
        pltpu.sync_copy(r_sh.at[pl.ds(sid * _RPS, _RPS)],
                        out_hbm.at[cid, pl.ds(sid * _RPS, _RPS)])

    return k(xa3, xb3, srcg, dstg)


def _degree(dstg):
    @functools.partial(
        pl.kernel,
        out_type=jax.ShapeDtypeStruct((_NC, _NP, 16), jnp.float32),
        mesh=_sc_mesh(),
        scratch_types=[
            pltpu.VMEM((_NCHD, _C), jnp.int32),
            pltpu.VMEM((_C, 16), jnp.float32),
            pltpu.VMEM((_RPS, 16), jnp.float32),
            pltpu.VMEM_SHARED((_NP, 16), jnp.float32),
        ],
        compiler_params=pltpu.CompilerParams(use_tc_tiling_on_sc=False),
    )
    def k(dst_hbm, out_hbm, dst_v, ones, zbuf, d_sh):
        cid = lax.axis_index("c")
        sid = lax.axis_index("s")
        wid = sid * _NC + cid

        pltpu.sync_copy(dst_hbm.at[wid], dst_v)

        def fill(r, carry):
            ones[r, pl.ds(0, 16)] = jnp.full((16,), 1.0, jnp.float32)
            return carry
        lax.fori_loop(0, _C, fill, 0)

        def zstore(r, carry):
            zbuf[r, pl.ds(0, 16)] = jnp.zeros((16,), jnp.float32)
            return carry
        lax.fori_loop(0, _RPS, zstore, 0)
        pltpu.sync_copy(zbuf, d_sh.at[pl.ds(sid * _RPS, _RPS)])
        plsc.subcore_barrier()

        def chunk(j, carry):
            pltpu.sync_copy(ones, d_sh.at[dst_v.at[j]], add=True)
            return carry
        lax.fori_loop(0, _NCHD, chunk, 0)

        plsc.subcore_barrier()
        pltpu.sync_copy(d_sh.at[pl.ds(sid * _RPS, _RPS)],
                        out_hbm.at[cid, pl.ds(sid * _RPS, _RPS)])

    return k(dstg)


def _full(shape):
    return pl.BlockSpec(shape, lambda i: (0,) * len(shape))


def _blk(w=_H):
    return pl.BlockSpec((_BN, w), lambda i: (i, 0))


def _encode(nf, nW1, nb1, nW2, nb2, a0, b0, mb0):
    nd = nf.shape[1]

    def body(nf_r, w1, b1, w2, b2, a_r, b_r, mb, x_o, xa_o, xb_o):
        x = (jnp.maximum(nf_r[...] @ w1[...] + b1[...], 0.0)
             @ w2[...] + b2[...])
        x_o[...] = x
        for c in range(_NC):
            xa_o[c] = x @ a_r[c]
            xb_o[c] = x @ b_r[c] + mb[c]

    o3 = jax.ShapeDtypeStruct((_NC, _NP, _HH), jnp.float32)
    hblk = pl.BlockSpec((_NC, _BN, _HH), lambda i: (0, i, 0))
    return pl.pallas_call(
        body,
        grid=(_NP // _BN,),
        in_specs=[_blk(nd)] + [_full(w.shape) for w in
                               (nW1, nb1, nW2, nb2, a0, b0, mb0)],
        out_specs=[_blk(), hblk, hblk],
        out_shape=[jax.ShapeDtypeStruct((_NP, _H), jnp.float32), o3, o3],
    )(nf, nW1, nb1, nW2, nb2, a0, b0, mb0)


def _update(x, rl, rr, deg16, mw2l, mw2r, mb2i, uw1a, uw1b, ub1i, uW2i, ub2i,
            gi, bi, a_n, b_n, mb_n, rW1, rb1, rW2, rb2):
    nblk = _NP // _BN

    def body(x_r, rl_r, rr_r, dg_r, w2l, w2r, mb2, w1a, w1b, b1, w2, b2,
             gg, bb, a_r, b_r, mbn, q1, qb1, q2, qb2,
             x_o, xa_o, xb_o, o_r, acc):
        i = pl.program_id(0)
        x_v = x_r[...]
        cnt = dg_r[...][:, 0:1]
        agg = rl_r[...] @ w2l[...] + rr_r[...] @ w2r[...] + cnt * mb2[...]
        h = jnp.maximum(x_v @ w1a[...] + agg @ w1b[...] + b1[...], 0.0)
        u = h @ w2[...] + b2[...]
        y = x_v + u
        mu = jnp.mean(y, axis=-1, keepdims=True)
        var = jnp.mean((y - mu) ** 2, axis=-1, keepdims=True)
        xn = (y - mu) / jnp.sqrt(var + 1e-5) * gg[...] + bb[...]
        x_o[...] = xn
        for c in range(_NC):
            xa_o[c] = xn @ a_r[c]
            xb_o[c] = xn @ b_r[c] + mbn[c]

        rows = (i * _BN
                + jax.lax.broadcasted_iota(jnp.int32, (_BN, 1), 0))
        s = jnp.sum(jnp.where(rows < _N, xn, 0.0), axis=0, keepdims=True)
        s8 = jnp.broadcast_to(s, (8, _H))

        @pl.when(i == 0)
        def _():
            acc[...] = s8

        @pl.when(i > 0)
        def _():
            acc[...] = acc[...] + s8

        ge8 = acc[...] * (1.0 / _N)
        o_r[...] = jnp.maximum(ge8 @ q1[...] + qb1[...], 0.0) @ q2[...] + qb2[...]

    ws = (mw2l, mw2r, mb2i, uw1a, uw1b, ub1i, uW2i, ub2i, gi, bi,
          a_n, b_n, mb_n, rW1, rb1, rW2, rb2)
    o3 = jax.ShapeDtypeStruct((_NC, _NP, _HH), jnp.float32)
    hblk = pl.BlockSpec((_NC, _BN, _HH), lambda i: (0, i, 0))
    return pl.pallas_call(
        body,
        grid=(nblk,),
        in_specs=([_blk(), _blk(_HH), _blk(_HH), _blk(16)]
                  + [_full(w.shape) for w in ws]),
        out_specs=[_blk(), hblk, hblk,
                   pl.BlockSpec((8, _H), lambda i: (0, 0))],
        out_shape=[jax.ShapeDtypeStruct((_NP, _H), jnp.float32), o3, o3,
                   jax.ShapeDtypeStruct((8, _H), jnp.float32)],
        scratch_shapes=[pltpu.VMEM((8, _H), jnp.float32)],
    )(x, rl, rr, deg16, *ws)


def kernel(node_features, edge_index, edge_features, nW1, nb1, nW2, nb2,
           eW1, eb1, eW2, eb2, mW1, mb1, mW2, mb2, uW1, ub1, uW2, ub2,
           g, bta, rW1, rb1, rW2, rb2):
    del edge_features, eW1, eb1, eW2, eb2

    nf = jnp.pad(node_features, ((0, _NP - _N), (0, 0)))
    srcg = edge_index[0].reshape(_NS, _NCH, _C)
    dstg = edge_index[1].reshape(_NS, _NCH, _C)
    dstw = edge_index[1].reshape(_NW, _NCHD, _C)

    def row(v):
        return v.reshape(1, -1)

    deg2 = _degree(dstw)
    deg16 = deg2[0] + deg2[1]

    def csplit(w):
        return w.reshape(w.shape[0], w.shape[1], _NC, _HH).transpose(0, 2, 1, 3)

    a_c = csplit(mW1[:, :_H, :])
    b_c = csplit(mW1[:, _H:, :])
    mb_c = csplit(mb1[:, None, :])

    x, xa3, xb3 = _encode(nf, nW1, row(nb1), nW2, row(nb2),
                          a_c[0], b_c[0], mb_c[0])

    def shift(w):
        return jnp.concatenate([w[1:], w[-1:]], axis=0)

    xs = (mW2[:, :_HH, :], mW2[:, _HH:, :], mb2[:, None, :],
          uW1[:, :_H, :], uW1[:, _H:, :], ub1[:, None, :],
          uW2, ub2[:, None, :], g[:, None, :], bta[:, None, :],
          shift(a_c), shift(b_c), shift(mb_c))

    rb1r, rb2r = row(rb1), row(rb2)
    out0 = jnp.zeros((8, _H), jnp.float32)

    def step(carry, ws):
        x_c, xa_c, xb_c, _ = carry
        (w2l, w2r, mb2_i, u1a, u1b, ub1_i, uw2_i, ub2_i, g_i, bta_i,
         a_n, b_n, mbn) = ws
        r2 = _edge_pass(xa_c, xb_c, srcg, dstg)
        x_n, xa_n, xb_n, o8 = _update(x_c, r2[0], r2[1], deg16, w2l, w2r,
                                      mb2_i, u1a, u1b, ub1_i, uw2_i, ub2_i,
                                      g_i, bta_i, a_n, b_n, mbn,
                                      rW1, rb1r, rW2, rb2r)
        return (x_n, xa_n, xb_n, o8), None

    (_, _, _, out8), _ = lax.scan(step, (x, xa3, xb3, out0), xs)
    return out8[0]


# speedup vs baseline: 9.0595x; 1.0065x over previous
import functools

import jax
import jax.numpy as jnp
from jax import lax
from jax.experimental import pallas as pl
from jax.experimental.pallas import tpu as pltpu
from jax.experimental.pallas import tpu_sc as plsc

_N = 10000
_E = 320000
_H = 128
_HH = _H // 2
_L = 3

_NC = 2
_NS = 16
_NW = _NC * _NS
_C = 80

_EPT = _E // _NS
_NCH = _EPT // _C
_EPW = _E // _NW
_NCHD = _EPW // _C

_NB = 4
_NP = 10112
_RPS = _NP // _NS

_BN = 1264


def _sc_mesh():
    return plsc.VectorSubcoreMesh(
        core_axis_name="c", subcore_axis_name="s",
        num_cores=_NC, num_subcores=_NS)


def _edge_pass(xa3, xb3, srcg, dstg):
    @functools.partial(
        pl.kernel,
        out_type=jax.ShapeDtypeStruct((_NC, _NP, _HH), jnp.float32),
        mesh=_sc_mesh(),
        scratch_types=[
            pltpu.VMEM((_NCH, _C), jnp.int32),
            pltpu.VMEM((_NCH, _C), jnp.int32),
            pltpu.VMEM((_C, _HH), jnp.float32),
            pltpu.VMEM((_C, _HH), jnp.float32),
            pltpu.VMEM((_C, _HH), jnp.float32),
            pltpu.VMEM((_C, _HH), jnp.float32),
            pltpu.VMEM((_C, _HH), jnp.float32),
            pltpu.VMEM((_C, _HH), jnp.float32),
            pltpu.VMEM((_C, _HH), jnp.float32),
            pltpu.VMEM((_C, _HH), jnp.float32),
            pltpu.VMEM_SHARED((_NP, _HH), jnp.float32),
            pltpu.SemaphoreType.DMA,
            pltpu.SemaphoreType.DMA,
            pltpu.SemaphoreType.DMA,
            pltpu.SemaphoreType.DMA,
            pltpu.SemaphoreType.DMA,
            pltpu.SemaphoreType.DMA,
            pltpu.SemaphoreType.DMA,
            pltpu.SemaphoreType.DMA,
        ],
        compiler_params=pltpu.CompilerParams(use_tc_tiling_on_sc=False),
    )
    def k(xa_hbm, xb_hbm, src_hbm, dst_hbm, out_hbm,
          src_v, dst_v, ar0, ar1, ar2, ar3, br0, br1, br2, br3, r_sh,
          sg0, sg1, sg2, sg3, ss0, ss1, ss2, ss3):
        cid = lax.axis_index("c")
        sid = lax.axis_index("s")
        ar = [ar0, ar1, ar2, ar3]
        br = [br0, br1, br2, br3]
        sg = [sg0, sg1, sg2, sg3]
        ss = [ss0, ss1, ss2, ss3]

        pltpu.sync_copy(src_hbm.at[sid], src_v)
        pltpu.sync_copy(dst_hbm.at[sid], dst_v)

        def zstore(t, carry):
            r = t // (_HH // 16)
            cc = (t % (_HH // 16)) * 16
            ar0[r, pl.ds(cc, 16)] = jnp.zeros((16,), jnp.float32)
            return carry
        lax.fori_loop(0, _C * (_HH // 16), zstore, 0)
        for t in range(_RPS // _C):
            pltpu.sync_copy(ar0, r_sh.at[pl.ds(sid * _RPS + t * _C, _C)])
        _RT = _RPS % _C
        pltpu.sync_copy(
            ar0.at[pl.ds(0, _RT)],
            r_sh.at[pl.ds(sid * _RPS + (_RPS // _C) * _C, _RT)])
        plsc.subcore_barrier()

        def gissue(j, b):
            pltpu.async_copy(xa_hbm.at[cid].at[src_v.at[j]], ar[b], sg[b])
            pltpu.async_copy(xb_hbm.at[cid].at[dst_v.at[j]], br[b], sg[b])

        def gwait(j, b):
            pltpu.make_async_copy(xa_hbm.at[cid].at[src_v.at[j]],
                                  ar[b], sg[b]).wait()
            pltpu.make_async_copy(xb_hbm.at[cid].at[dst_v.at[j]],
                                  br[b], sg[b]).wait()

        def swait(j, b):
            pltpu.make_async_copy(ar[b], r_sh.at[dst_v.at[j]], ss[b]).wait()

        gissue(0, 0)
        gissue(1, 1)
        gissue(2, 2)

        def body(j, b):
            bn = (b + 3) % 4

            @pl.when(jnp.logical_and(j >= 1, j + 3 < _NCH))
            def _():
                swait(j - 1, bn)

            @pl.when(j + 3 < _NCH)
            def _():
                gissue(j + 3, bn)

            gwait(j, b)

            def rows(r8, rc):
                for rr in range(8):
                    for cc in range(_HH // 16):
                        sl = pl.ds(cc * 16, 16)
                        r = r8 * 8 + rr
                        ar[b][r, sl] = jnp.maximum(
                            ar[b][r, sl] + br[b][r, sl], 0.0)
                return rc
            lax.fori_loop(0, _C // 8, rows, 0)

            pltpu.async_copy(ar[b], r_sh.at[dst_v.at[j]], ss[b],
                             add=True)

        def round_(j0, carry):
            j4 = j0 * 4
            for b in range(4):
                body(j4 + b, b)
            return carry
        lax.fori_loop(0, _NCH // 4, round_, 0)
        for j in range(_NCH - _NCH % 4, _NCH):
            body(j, j % 4)

        for b in range(4):
            swait(_NCH - 4 + b, (_NCH - 4 + b) % 4)

        plsc.subcore_barrier()
        pltpu.sync_copy(r_sh.at[pl.ds(sid * _RPS, _RPS)],
                        out_hbm.at[cid, pl.ds(sid * _RPS, _RPS)])

    return k(xa3, xb3, srcg, dstg)


def _degree(dstg):
    @functools.partial(
        pl.kernel,
        out_type=jax.ShapeDtypeStruct((_NC, _NP, 16), jnp.float32),
        mesh=_sc_mesh(),
        scratch_types=[
            pltpu.VMEM((_NCHD, _C), jnp.int32),
            pltpu.VMEM((_C, 16), jnp.float32),
            pltpu.VMEM((_RPS, 16), jnp.float32),
            pltpu.VMEM_SHARED((_NP, 16), jnp.float32),
        ],
        compiler_params=pltpu.CompilerParams(use_tc_tiling_on_sc=False),
    )
    def k(dst_hbm, out_hbm, dst_v, ones, zbuf, d_sh):
        cid = lax.axis_index("c")
        sid = lax.axis_index("s")
        wid = sid * _NC + cid

        pltpu.sync_copy(dst_hbm.at[wid], dst_v)

        def fill(r, carry):
            ones[r, pl.ds(0, 16)] = jnp.full((16,), 1.0, jnp.float32)
            return carry
        lax.fori_loop(0, _C, fill, 0)

        def zstore(r, carry):
            zbuf[r, pl.ds(0, 16)] = jnp.zeros((16,), jnp.float32)
            return carry
        lax.fori_loop(0, _RPS, zstore, 0)
        pltpu.sync_copy(zbuf, d_sh.at[pl.ds(sid * _RPS, _RPS)])
        plsc.subcore_barrier()

        def chunk(j, carry):
            pltpu.sync_copy(ones, d_sh.at[dst_v.at[j]], add=True)
            return carry
        lax.fori_loop(0, _NCHD, chunk, 0)

        plsc.subcore_barrier()
        pltpu.sync_copy(d_sh.at[pl.ds(sid * _RPS, _RPS)],
                        out_hbm.at[cid, pl.ds(sid * _RPS, _RPS)])

    return k(dstg)


def _full(shape):
    return pl.BlockSpec(shape, lambda i: (0,) * len(shape))


def _blk(w=_H):
    return pl.BlockSpec((_BN, w), lambda i: (i, 0))


def _encode(nf, nW1, nb1, nW2, nb2, a0, b0, mb0):
    nd = nf.shape[1]

    def body(nf_r, w1, b1, w2, b2, a_r, b_r, mb, x_o, xa_o, xb_o):
        x = (jnp.maximum(nf_r[...] @ w1[...] + b1[...], 0.0)
             @ w2[...] + b2[...])
        x_o[...] = x
        for c in range(_NC):
            xa_o[c] = x @ a_r[c]
            xb_o[c] = x @ b_r[c] + mb[c]

    o3 = jax.ShapeDtypeStruct((_NC, _NP, _HH), jnp.float32)
    hblk = pl.BlockSpec((_NC, _BN, _HH), lambda i: (0, i, 0))
    return pl.pallas_call(
        body,
        grid=(_NP // _BN,),
        in_specs=[_blk(nd)] + [_full(w.shape) for w in
                               (nW1, nb1, nW2, nb2, a0, b0, mb0)],
        out_specs=[_blk(), hblk, hblk],
        out_shape=[jax.ShapeDtypeStruct((_NP, _H), jnp.float32), o3, o3],
    )(nf, nW1, nb1, nW2, nb2, a0, b0, mb0)


def _update(x, rl, rr, deg16, mw2l, mw2r, mb2i, uw1a, uw1b, ub1i, uW2i, ub2i,
            gi, bi, a_n, b_n, mb_n, rW1, rb1, rW2, rb2):
    nblk = _NP // _BN

    def body(x_r, rl_r, rr_r, dg_r, w2l, w2r, mb2, w1a, w1b, b1, w2, b2,
             gg, bb, a_r, b_r, mbn, q1, qb1, q2, qb2,
             x_o, xa_o, xb_o, o_r, acc):
        i = pl.program_id(0)
        x_v = x_r[...]
        cnt = dg_r[...][:, 0:1]
        agg = rl_r[...] @ w2l[...] + rr_r[...] @ w2r[...] + cnt * mb2[...]
        h = jnp.maximum(x_v @ w1a[...] + agg @ w1b[...] + b1[...], 0.0)
        u = h @ w2[...] + b2[...]
        y = x_v + u
        mu = jnp.mean(y, axis=-1, keepdims=True)
        var = jnp.mean((y - mu) ** 2, axis=-1, keepdims=True)
        xn = (y - mu) / jnp.sqrt(var + 1e-5) * gg[...] + bb[...]
        x_o[...] = xn
        for c in range(_NC):
            xa_o[c] = xn @ a_r[c]
            xb_o[c] = xn @ b_r[c] + mbn[c]

        rows = (i * _BN
                + jax.lax.broadcasted_iota(jnp.int32, (_BN, 1), 0))
        s = jnp.sum(jnp.where(rows < _N, xn, 0.0), axis=0, keepdims=True)
        s8 = jnp.broadcast_to(s, (8, _H))

        @pl.when(i == 0)
        def _():
            acc[...] = s8

        @pl.when(i > 0)
        def _():
            acc[...] = acc[...] + s8

        ge8 = acc[...] * (1.0 / _N)
        o_r[...] = jnp.maximum(ge8 @ q1[...] + qb1[...], 0.0) @ q2[...] + qb2[...]

    ws = (mw2l, mw2r, mb2i, uw1a, uw1b, ub1i, uW2i, ub2i, gi, bi,
          a_n, b_n, mb_n, rW1, rb1, rW2, rb2)
    o3 = jax.ShapeDtypeStruct((_NC, _NP, _HH), jnp.float32)
    hblk = pl.BlockSpec((_NC, _BN, _HH), lambda i: (0, i, 0))
    return pl.pallas_call(
        body,
        grid=(nblk,),
        in_specs=([_blk(), _blk(_HH), _blk(_HH), _blk(16)]
                  + [_full(w.shape) for w in ws]),
        out_specs=[_blk(), hblk, hblk,
                   pl.BlockSpec((8, _H), lambda i: (0, 0))],
        out_shape=[jax.ShapeDtypeStruct((_NP, _H), jnp.float32), o3, o3,
                   jax.ShapeDtypeStruct((8, _H), jnp.float32)],
        scratch_shapes=[pltpu.VMEM((8, _H), jnp.float32)],
    )(x, rl, rr, deg16, *ws)


def kernel(node_features, edge_index, edge_features, nW1, nb1, nW2, nb2,
           eW1, eb1, eW2, eb2, mW1, mb1, mW2, mb2, uW1, ub1, uW2, ub2,
           g, bta, rW1, rb1, rW2, rb2):
    del edge_features, eW1, eb1, eW2, eb2

    nf = jnp.pad(node_features, ((0, _NP - _N), (0, 0)))
    srcg = edge_index[0].reshape(_NS, _NCH, _C)
    dstg = edge_index[1].reshape(_NS, _NCH, _C)
    dstw = edge_index[1].reshape(_NW, _NCHD, _C)

    def row(v):
        return v.reshape(1, -1)

    deg2 = _degree(dstw)
    deg16 = deg2[0] + deg2[1]

    def csplit(w):
        return w.reshape(w.shape[0], w.shape[1], _NC, _HH).transpose(0, 2, 1, 3)

    a_c = csplit(mW1[:, :_H, :])
    b_c = csplit(mW1[:, _H:, :])
    mb_c = csplit(mb1[:, None, :])

    x, xa3, xb3 = _encode(nf, nW1, row(nb1), nW2, row(nb2),
                          a_c[0], b_c[0], mb_c[0])

    def shift(w):
        return jnp.concatenate([w[1:], w[-1:]], axis=0)

    xs = (mW2[:, :_HH, :], mW2[:, _HH:, :], mb2[:, None, :],
          uW1[:, :_H, :], uW1[:, _H:, :], ub1[:, None, :],
          uW2, ub2[:, None, :], g[:, None, :], bta[:, None, :],
          shift(a_c), shift(b_c), shift(mb_c))

    rb1r, rb2r = row(rb1), row(rb2)
    out0 = jnp.zeros((8, _H), jnp.float32)

    def step(carry, ws):
        x_c, xa_c, xb_c, _ = carry
        (w2l, w2r, mb2_i, u1a, u1b, ub1_i, uw2_i, ub2_i, g_i, bta_i,
         a_n, b_n, mbn) = ws
        r2 = _edge_pass(xa_c, xb_c, srcg, dstg)
        x_n, xa_n, xb_n, o8 = _update(x_c, r2[0], r2[1], deg16, w2l, w2r,
                                      mb2_i, u1a, u1b, ub1_i, uw2_i, ub2_i,
                                      g_i, bta_i, a_n, b_n, mbn,
                                      rW1, rb1r, rW2, rb2r)
        return (x_n, xa_n, xb_n, o8), None

    (_, _, _, out8), _ = lax.scan(step, (x, xa3, xb3, out0), xs)
    return out8[0]
